# baseline (speedup 1.0000x reference)
import functools

import jax
import jax.numpy as jnp
from jax import lax
from jax.experimental import pallas as pl
from jax.experimental.pallas import tpu as pltpu, tpu_sc as plsc

N_NODE = 10000
NPAD = 10240
EMB = 112
DPAD = 128
DH = DPAD // 2
N_EDGE = 640000
NC = 2
NS = 16
L = 16
EPS = N_EDGE // NS
B = 80
NCH = EPS // B
RPT = NPAD // NS
SBUF = RPT // 2


NBUF = 4


def _body(xh, rows, cols, w, out, x1, xnew,
          colsv0, rowsv0, wv0, G0, colsv1, rowsv1, wv1, G1,
          colsv2, rowsv2, wv2, G2, colsv3, rowsv3, wv3, G3,
          bufA, bufB,
          semi0, semi1, semi2, semi3,
          semg0, semg1, semg2, semg3,
          sems0, sems1, sems2, sems3):
    cid = lax.axis_index("c")
    sid = lax.axis_index("s")
    half_base = cid * NPAD
    stripe = half_base + sid * RPT
    sstripe = sid * RPT
    ebase = sid * EPS

    bufs = ((colsv0, rowsv0, wv0, G0, semi0, semg0, sems0),
            (colsv1, rowsv1, wv1, G1, semi1, semg1, sems1),
            (colsv2, rowsv2, wv2, G2, semi2, semg2, sems2),
            (colsv3, rowsv3, wv3, G3, semi3, semg3, sems3))

    def zero_bufA():
        zz = jnp.zeros((L,), jnp.float32)
        def zb(i, c):
            for j in range(DH // L):
                bufA[i, pl.ds(j * L, L)] = zz
            return c
        lax.fori_loop(0, SBUF, zb, 0)

    def zero_xnew():
        for p in range(RPT // SBUF):
            pltpu.sync_copy(bufA, xnew.at[pl.ds(sstripe + p * SBUF, SBUF)])

    def start_idx(ci, p):
        colsv, rowsv, wv, _, semi, _, _ = bufs[p]
        base = ebase + ci * B
        pltpu.async_copy(cols.at[pl.ds(base, B)], colsv, semi)
        pltpu.async_copy(rows.at[pl.ds(base, B)], rowsv, semi)
        pltpu.async_copy(w.at[pl.ds(base, B)], wv, semi)

    def wait_idx(p):
        colsv, rowsv, wv, _, semi, _, _ = bufs[p]
        pltpu.make_async_copy(cols.at[pl.ds(0, B)], colsv, semi).wait()
        pltpu.make_async_copy(rows.at[pl.ds(0, B)], rowsv, semi).wait()
        pltpu.make_async_copy(w.at[pl.ds(0, B)], wv, semi).wait()

    def start_gather(src_hbm, p):
        colsv, _, _, G, _, semg, _ = bufs[p]
        wait_idx(p)
        for j in range(B // L):
            s = pl.ds(j * L, L)
            colsv[s] = colsv[s] + half_base
        pltpu.async_copy(src_hbm.at[colsv], G, semg)

    def wait_gather(src_hbm, p):
        colsv, _, _, G, _, semg, _ = bufs[p]
        pltpu.make_async_copy(src_hbm.at[colsv], G, semg).wait()

    def process(src_hbm, p):
        _, rowsv, wv, G, _, _, sems = bufs[p]
        wait_gather(src_hbm, p)
        def scale(g, c2):
            wv16 = wv[pl.ds(g * L, L)]
            for t in range(L):
                wvec = wv16[jnp.full((L,), t, jnp.int32)]
                e = g * L + t
                for j in range(DH // L):
                    s = pl.ds(j * L, L)
                    G[e, s] = G[e, s] * wvec
            return c2
        lax.fori_loop(0, B // L, scale, 0)
        pltpu.async_copy(G, xnew.at[rowsv], sems, add=True)

    def wait_scatter(p):
        _, rowsv, _, G, _, _, sems = bufs[p]
        pltpu.make_async_copy(G, xnew.at[rowsv], sems).wait()

    def edge_pass(src_hbm):
        start_idx(0, 0)
        start_idx(1, 1)
        start_gather(src_hbm, 0)
        start_gather(src_hbm, 1)
        process(src_hbm, 0)
        start_idx(2, 2)
        start_gather(src_hbm, 2)
        process(src_hbm, 1)
        start_idx(3, 3)

        def step(k, c):
            cb = 4 * k + 2
            for d in range(NBUF):
                p0 = (2 + d) % NBUF
                p1 = (3 + d) % NBUF
                p2 = (d) % NBUF
                start_gather(src_hbm, p1)
                process(src_hbm, p0)
                wait_scatter(p2)
                start_idx(cb + d + 2, p2)
            return c
        lax.fori_loop(0, (NCH - 4) // NBUF, step, 0)
        start_gather(src_hbm, (NCH - 1) % NBUF)
        process(src_hbm, (NCH - 2) % NBUF)
        wait_scatter(NCH % NBUF)
        process(src_hbm, (NCH - 1) % NBUF)
        wait_scatter((NCH + 1) % NBUF)
        wait_scatter((NCH - 2) % NBUF)
        wait_scatter((NCH - 1) % NBUF)

    zero_bufA()
    zero_xnew()
    plsc.subcore_barrier()
    edge_pass(xh)
    plsc.subcore_barrier()

    for p in range(RPT // SBUF):
        pltpu.sync_copy(xnew.at[pl.ds(sstripe + p * SBUF, SBUF)], bufB)
        pltpu.sync_copy(bufB, x1.at[pl.ds(stripe + p * SBUF, SBUF)])
    zero_xnew()
    plsc.subcore_barrier()

    edge_pass(x1)
    plsc.subcore_barrier()

    for p in range(RPT // SBUF):
        pltpu.sync_copy(xh.at[pl.ds(stripe + p * SBUF, SBUF)], bufA)
        pltpu.sync_copy(x1.at[pl.ds(stripe + p * SBUF, SBUF)], bufB)
        def addb(i, c):
            for j in range(DH // L):
                s = pl.ds(j * L, L)
                bufA[i, s] = bufA[i, s] + bufB[i, s]
            return c
        lax.fori_loop(0, SBUF, addb, 0)
        pltpu.sync_copy(xnew.at[pl.ds(sstripe + p * SBUF, SBUF)], bufB)
        def fin(i, c):
            for j in range(DH // L):
                s = pl.ds(j * L, L)
                bufA[i, s] = (bufA[i, s] + bufB[i, s]) * (1.0 / 3.0)
            return c
        lax.fori_loop(0, SBUF, fin, 0)
        pltpu.sync_copy(bufA, out.at[pl.ds(stripe + p * SBUF, SBUF)])


@jax.jit
def kernel(embedding, edge_index, edge_weight):
    xpad = jnp.pad(embedding, ((0, NPAD - N_NODE), (0, DPAD - EMB)))
    xh = jnp.concatenate([xpad[:, :DH], xpad[:, DH:]], axis=0)
    rows = edge_index[0]
    cols = edge_index[1]

    f32 = jnp.float32
    run = pl.kernel(
        _body,
        out_type=(
            jax.ShapeDtypeStruct((NC * NPAD, DH), f32),
            jax.ShapeDtypeStruct((NC * NPAD, DH), f32),
        ),
        mesh=plsc.VectorSubcoreMesh(
            core_axis_name="c", subcore_axis_name="s",
            num_cores=NC, num_subcores=NS),
        compiler_params=pltpu.CompilerParams(
            use_tc_tiling_on_sc=False, needs_layout_passes=False),
        scratch_types=(
            [pltpu.VMEM_SHARED((NPAD, DH), f32)]
            + [t for _ in range(NBUF)
               for t in (pltpu.VMEM((B,), jnp.int32),
                         pltpu.VMEM((B,), jnp.int32),
                         pltpu.VMEM((B,), f32),
                         pltpu.VMEM((B, DH), f32))]
            + [pltpu.VMEM((SBUF, DH), f32),
               pltpu.VMEM((SBUF, DH), f32)]
            + [pltpu.SemaphoreType.DMA] * (3 * NBUF)
        ),
    )
    o, _x1 = run(xh, rows, cols, edge_weight)
    return jnp.concatenate([o[:N_NODE], o[NPAD:NPAD + N_NODE]], axis=1)[:, :EMB]


# speedup vs baseline: 1.5316x; 1.5316x over previous
import functools

import jax
import jax.numpy as jnp
from jax import lax
from jax.experimental import pallas as pl
from jax.experimental.pallas import tpu as pltpu, tpu_sc as plsc

N_NODE = 10000
NPAD = 10240
EMB = 112
DPAD = 128
DH = DPAD // 2
N_EDGE = 640000
NC = 2
NS = 16
L = 16
EPS = N_EDGE // NS
B = 80
NCH = EPS // B
RPT = NPAD // NS
SBUF = RPT // 2


NBUF = 4


def _body(xh, rows, cols, w, out, x1, xnew,
          colsv0, rowsv0, wv0, G0, colsv1, rowsv1, wv1, G1,
          colsv2, rowsv2, wv2, G2, colsv3, rowsv3, wv3, G3,
          bufA, bufB,
          semi0, semi1, semi2, semi3,
          semg0, semg1, semg2, semg3,
          sems0, sems1, sems2, sems3):
    cid = lax.axis_index("c")
    sid = lax.axis_index("s")
    half_base = cid * NPAD
    stripe = half_base + sid * RPT
    sstripe = sid * RPT
    ebase = sid * EPS

    bufs = ((colsv0, rowsv0, wv0, G0, semi0, semg0, sems0),
            (colsv1, rowsv1, wv1, G1, semi1, semg1, sems1),
            (colsv2, rowsv2, wv2, G2, semi2, semg2, sems2),
            (colsv3, rowsv3, wv3, G3, semi3, semg3, sems3))

    def zero_bufA():
        zz = jnp.zeros((L,), jnp.float32)
        def zb(i, c):
            for j in range(DH // L):
                bufA[i, pl.ds(j * L, L)] = zz
            return c
        lax.fori_loop(0, SBUF, zb, 0)

    def zero_xnew():
        for p in range(RPT // SBUF):
            pltpu.sync_copy(bufA, xnew.at[pl.ds(sstripe + p * SBUF, SBUF)])

    def start_idx(ci, p):
        colsv, rowsv, wv, _, semi, _, _ = bufs[p]
        base = ebase + ci * B
        pltpu.async_copy(cols.at[pl.ds(base, B)], colsv, semi)
        pltpu.async_copy(rows.at[pl.ds(base, B)], rowsv, semi)
        pltpu.async_copy(w.at[pl.ds(base, B)], wv, semi)

    def wait_idx(p):
        colsv, rowsv, wv, _, semi, _, _ = bufs[p]
        pltpu.make_async_copy(cols.at[pl.ds(0, B)], colsv, semi).wait()
        pltpu.make_async_copy(rows.at[pl.ds(0, B)], rowsv, semi).wait()
        pltpu.make_async_copy(w.at[pl.ds(0, B)], wv, semi).wait()

    def start_gather(src_hbm, p):
        colsv, _, _, G, _, semg, _ = bufs[p]
        wait_idx(p)
        for j in range(B // L):
            s = pl.ds(j * L, L)
            colsv[s] = colsv[s] + half_base
        pltpu.async_copy(src_hbm.at[colsv], G, semg)

    def wait_gather(src_hbm, p):
        colsv, _, _, G, _, semg, _ = bufs[p]
        pltpu.make_async_copy(src_hbm.at[colsv], G, semg).wait()

    def process(src_hbm, p):
        _, rowsv, wv, G, _, _, sems = bufs[p]
        wait_gather(src_hbm, p)
        def scale(e, c2):
            idx = jnp.zeros((L,), jnp.int32) + e
            wvec = plsc.load_gather(wv, [idx])
            for j in range(DH // L):
                s = pl.ds(j * L, L)
                G[e, s] = G[e, s] * wvec
            return c2
        lax.fori_loop(0, B, scale, 0)
        pltpu.async_copy(G, xnew.at[rowsv], sems, add=True)

    def wait_scatter(p):
        _, rowsv, _, G, _, _, sems = bufs[p]
        pltpu.make_async_copy(G, xnew.at[rowsv], sems).wait()

    def edge_pass(src_hbm):
        start_idx(0, 0)
        start_idx(1, 1)
        start_gather(src_hbm, 0)
        start_gather(src_hbm, 1)
        process(src_hbm, 0)
        start_idx(2, 2)
        start_gather(src_hbm, 2)
        process(src_hbm, 1)
        start_idx(3, 3)

        def step(k, c):
            cb = 4 * k + 2
            for d in range(NBUF):
                p0 = (2 + d) % NBUF
                p1 = (3 + d) % NBUF
                p2 = (d) % NBUF
                start_gather(src_hbm, p1)
                process(src_hbm, p0)
                wait_scatter(p2)
                start_idx(cb + d + 2, p2)
            return c
        lax.fori_loop(0, (NCH - 4) // NBUF, step, 0)
        start_gather(src_hbm, (NCH - 1) % NBUF)
        process(src_hbm, (NCH - 2) % NBUF)
        wait_scatter(NCH % NBUF)
        process(src_hbm, (NCH - 1) % NBUF)
        wait_scatter((NCH + 1) % NBUF)
        wait_scatter((NCH - 2) % NBUF)
        wait_scatter((NCH - 1) % NBUF)

    zero_bufA()
    zero_xnew()
    plsc.subcore_barrier()
    edge_pass(xh)
    plsc.subcore_barrier()

    for p in range(RPT // SBUF):
        pltpu.sync_copy(xnew.at[pl.ds(sstripe + p * SBUF, SBUF)], bufB)
        pltpu.sync_copy(bufB, x1.at[pl.ds(stripe + p * SBUF, SBUF)])
    zero_xnew()
    plsc.subcore_barrier()

    edge_pass(x1)
    plsc.subcore_barrier()

    for p in range(RPT // SBUF):
        pltpu.sync_copy(xh.at[pl.ds(stripe + p * SBUF, SBUF)], bufA)
        pltpu.sync_copy(x1.at[pl.ds(stripe + p * SBUF, SBUF)], bufB)
        def addb(i, c):
            for j in range(DH // L):
                s = pl.ds(j * L, L)
                bufA[i, s] = bufA[i, s] + bufB[i, s]
            return c
        lax.fori_loop(0, SBUF, addb, 0)
        pltpu.sync_copy(xnew.at[pl.ds(sstripe + p * SBUF, SBUF)], bufB)
        def fin(i, c):
            for j in range(DH // L):
                s = pl.ds(j * L, L)
                bufA[i, s] = (bufA[i, s] + bufB[i, s]) * (1.0 / 3.0)
            return c
        lax.fori_loop(0, SBUF, fin, 0)
        pltpu.sync_copy(bufA, out.at[pl.ds(stripe + p * SBUF, SBUF)])


@jax.jit
def kernel(embedding, edge_index, edge_weight):
    xpad = jnp.pad(embedding, ((0, NPAD - N_NODE), (0, DPAD - EMB)))
    xh = jnp.concatenate([xpad[:, :DH], xpad[:, DH:]], axis=0)
    rows = edge_index[0]
    cols = edge_index[1]

    f32 = jnp.float32
    run = pl.kernel(
        _body,
        out_type=(
            jax.ShapeDtypeStruct((NC * NPAD, DH), f32),
            jax.ShapeDtypeStruct((NC * NPAD, DH), f32),
        ),
        mesh=plsc.VectorSubcoreMesh(
            core_axis_name="c", subcore_axis_name="s",
            num_cores=NC, num_subcores=NS),
        compiler_params=pltpu.CompilerParams(
            use_tc_tiling_on_sc=False, needs_layout_passes=False),
        scratch_types=(
            [pltpu.VMEM_SHARED((NPAD, DH), f32)]
            + [t for _ in range(NBUF)
               for t in (pltpu.VMEM((B,), jnp.int32),
                         pltpu.VMEM((B,), jnp.int32),
                         pltpu.VMEM((B,), f32),
                         pltpu.VMEM((B, DH), f32))]
            + [pltpu.VMEM((SBUF, DH), f32),
               pltpu.VMEM((SBUF, DH), f32)]
            + [pltpu.SemaphoreType.DMA] * (3 * NBUF)
        ),
    )
    o, _x1 = run(xh, rows, cols, edge_weight)
    return jnp.concatenate([o[:N_NODE], o[NPAD:NPAD + N_NODE]], axis=1)[:, :EMB]


# speedup vs baseline: 1.6324x; 1.0658x over previous
import functools

import jax
import jax.numpy as jnp
from jax import lax
from jax.experimental import pallas as pl
from jax.experimental.pallas import tpu as pltpu, tpu_sc as plsc

N_NODE = 10000
NPAD = 10240
EMB = 112
DPAD = 128
DH = DPAD // 2
N_EDGE = 640000
NC = 2
NS = 16
L = 16
EPS = N_EDGE // NS
B = 80
NCH = EPS // B
RPT = NPAD // NS
SBUF = RPT // 2


NBUF = 4


def _body(xh, rows, cols, w, out, x1, xnew,
          colsv0, rowsv0, wv0, G0, colsv1, rowsv1, wv1, G1,
          colsv2, rowsv2, wv2, G2, colsv3, rowsv3, wv3, G3,
          bufA, bufB,
          semi0, semi1, semi2, semi3,
          semg0, semg1, semg2, semg3,
          sems0, sems1, sems2, sems3):
    cid = lax.axis_index("c")
    sid = lax.axis_index("s")
    half_base = cid * NPAD
    stripe = half_base + sid * RPT
    sstripe = sid * RPT
    ebase = sid * EPS

    bufs = ((colsv0, rowsv0, wv0, G0, semi0, semg0, sems0),
            (colsv1, rowsv1, wv1, G1, semi1, semg1, sems1),
            (colsv2, rowsv2, wv2, G2, semi2, semg2, sems2),
            (colsv3, rowsv3, wv3, G3, semi3, semg3, sems3))

    def zero_bufA():
        zz = jnp.zeros((L,), jnp.float32)
        def zb(i, c):
            for j in range(DH // L):
                bufA[i, pl.ds(j * L, L)] = zz
            return c
        lax.fori_loop(0, SBUF, zb, 0)

    def zero_xnew():
        for p in range(RPT // SBUF):
            pltpu.sync_copy(bufA, xnew.at[pl.ds(sstripe + p * SBUF, SBUF)])

    def start_idx(ci, p):
        colsv, rowsv, wv, _, semi, _, _ = bufs[p]
        base = ebase + ci * B
        pltpu.async_copy(cols.at[pl.ds(base, B)], colsv, semi)
        pltpu.async_copy(rows.at[pl.ds(base, B)], rowsv, semi)
        pltpu.async_copy(w.at[pl.ds(base, B)], wv, semi)

    def wait_idx(p):
        colsv, rowsv, wv, _, semi, _, _ = bufs[p]
        pltpu.make_async_copy(cols.at[pl.ds(0, B)], colsv, semi).wait()
        pltpu.make_async_copy(rows.at[pl.ds(0, B)], rowsv, semi).wait()
        pltpu.make_async_copy(w.at[pl.ds(0, B)], wv, semi).wait()

    def start_gather(src_hbm, p):
        colsv, _, _, G, _, semg, _ = bufs[p]
        wait_idx(p)
        for j in range(B // L):
            s = pl.ds(j * L, L)
            colsv[s] = colsv[s] + half_base
        pltpu.async_copy(src_hbm.at[colsv], G, semg)

    def wait_gather(src_hbm, p):
        colsv, _, _, G, _, semg, _ = bufs[p]
        pltpu.make_async_copy(src_hbm.at[colsv], G, semg).wait()

    def process(src_hbm, p):
        _, rowsv, wv, G, _, _, sems = bufs[p]
        wait_gather(src_hbm, p)
        UNR = 4
        def scale(i, c2):
            e0 = i * UNR
            base = jnp.zeros((L,), jnp.int32) + e0
            for t in range(UNR):
                wvec = plsc.load_gather(wv, [base + t])
                for j in range(DH // L):
                    s = pl.ds(j * L, L)
                    G[e0 + t, s] = G[e0 + t, s] * wvec
            return c2
        lax.fori_loop(0, B // UNR, scale, 0)
        pltpu.async_copy(G, xnew.at[rowsv], sems, add=True)

    def wait_scatter(p):
        _, rowsv, _, G, _, _, sems = bufs[p]
        pltpu.make_async_copy(G, xnew.at[rowsv], sems).wait()

    def edge_pass(src_hbm):
        start_idx(0, 0)
        start_idx(1, 1)
        start_gather(src_hbm, 0)
        start_gather(src_hbm, 1)
        process(src_hbm, 0)
        start_idx(2, 2)
        start_gather(src_hbm, 2)
        process(src_hbm, 1)
        start_idx(3, 3)

        def step(k, c):
            cb = 4 * k + 2
            for d in range(NBUF):
                p0 = (2 + d) % NBUF
                p1 = (3 + d) % NBUF
                p2 = (d) % NBUF
                start_gather(src_hbm, p1)
                process(src_hbm, p0)
                wait_scatter(p2)
                start_idx(cb + d + 2, p2)
            return c
        lax.fori_loop(0, (NCH - 4) // NBUF, step, 0)
        start_gather(src_hbm, (NCH - 1) % NBUF)
        process(src_hbm, (NCH - 2) % NBUF)
        wait_scatter(NCH % NBUF)
        process(src_hbm, (NCH - 1) % NBUF)
        wait_scatter((NCH + 1) % NBUF)
        wait_scatter((NCH - 2) % NBUF)
        wait_scatter((NCH - 1) % NBUF)

    zero_bufA()
    zero_xnew()
    plsc.subcore_barrier()
    edge_pass(xh)
    plsc.subcore_barrier()

    for p in range(RPT // SBUF):
        pltpu.sync_copy(xnew.at[pl.ds(sstripe + p * SBUF, SBUF)], bufB)
        pltpu.sync_copy(bufB, x1.at[pl.ds(stripe + p * SBUF, SBUF)])
    zero_xnew()
    plsc.subcore_barrier()

    edge_pass(x1)
    plsc.subcore_barrier()

    for p in range(RPT // SBUF):
        pltpu.sync_copy(xh.at[pl.ds(stripe + p * SBUF, SBUF)], bufA)
        pltpu.sync_copy(x1.at[pl.ds(stripe + p * SBUF, SBUF)], bufB)
        def addb(i, c):
            for j in range(DH // L):
                s = pl.ds(j * L, L)
                bufA[i, s] = bufA[i, s] + bufB[i, s]
            return c
        lax.fori_loop(0, SBUF, addb, 0)
        pltpu.sync_copy(xnew.at[pl.ds(sstripe + p * SBUF, SBUF)], bufB)
        def fin(i, c):
            for j in range(DH // L):
                s = pl.ds(j * L, L)
                bufA[i, s] = (bufA[i, s] + bufB[i, s]) * (1.0 / 3.0)
            return c
        lax.fori_loop(0, SBUF, fin, 0)
        pltpu.sync_copy(bufA, out.at[pl.ds(stripe + p * SBUF, SBUF)])


@jax.jit
def kernel(embedding, edge_index, edge_weight):
    xpad = jnp.pad(embedding, ((0, NPAD - N_NODE), (0, DPAD - EMB)))
    xh = jnp.concatenate([xpad[:, :DH], xpad[:, DH:]], axis=0)
    rows = edge_index[0]
    cols = edge_index[1]

    f32 = jnp.float32
    run = pl.kernel(
        _body,
        out_type=(
            jax.ShapeDtypeStruct((NC * NPAD, DH), f32),
            jax.ShapeDtypeStruct((NC * NPAD, DH), f32),
        ),
        mesh=plsc.VectorSubcoreMesh(
            core_axis_name="c", subcore_axis_name="s",
            num_cores=NC, num_subcores=NS),
        compiler_params=pltpu.CompilerParams(
            use_tc_tiling_on_sc=False, needs_layout_passes=False),
        scratch_types=(
            [pltpu.VMEM_SHARED((NPAD, DH), f32)]
            + [t for _ in range(NBUF)
               for t in (pltpu.VMEM((B,), jnp.int32),
                         pltpu.VMEM((B,), jnp.int32),
                         pltpu.VMEM((B,), f32),
                         pltpu.VMEM((B, DH), f32))]
            + [pltpu.VMEM((SBUF, DH), f32),
               pltpu.VMEM((SBUF, DH), f32)]
            + [pltpu.SemaphoreType.DMA] * (3 * NBUF)
        ),
    )
    o, _x1 = run(xh, rows, cols, edge_weight)
    return jnp.concatenate([o[:N_NODE], o[NPAD:NPAD + N_NODE]], axis=1)[:, :EMB]


# speedup vs baseline: 1.8647x; 1.1423x over previous
import functools

import jax
import jax.numpy as jnp
from jax import lax
from jax.experimental import pallas as pl
from jax.experimental.pallas import tpu as pltpu, tpu_sc as plsc

N_NODE = 10000
NPAD = 10240
EMB = 112
DPAD = 128
DH = DPAD // 2
N_EDGE = 640000
NC = 2
NS = 16
L = 16
EPS = N_EDGE // NS
B = 80
NCH = EPS // B
RPT = NPAD // NS
SBUF = RPT // 2


NBUF = 4


def _body(xh, rows, cols, w, out, x1, xnew,
          colsv0, rowsv0, wv0, G0, M0, colsv1, rowsv1, wv1, G1, M1,
          colsv2, rowsv2, wv2, G2, M2, colsv3, rowsv3, wv3, G3, M3,
          bufA, bufB,
          semi0, semi1, semi2, semi3,
          semg0, semg1, semg2, semg3,
          sems0, sems1, sems2, sems3):
    cid = lax.axis_index("c")
    sid = lax.axis_index("s")
    half_base = cid * NPAD
    stripe = half_base + sid * RPT
    sstripe = sid * RPT
    ebase = sid * EPS

    bufs = ((colsv0, rowsv0, wv0, G0, M0, semi0, semg0, sems0),
            (colsv1, rowsv1, wv1, G1, M1, semi1, semg1, sems1),
            (colsv2, rowsv2, wv2, G2, M2, semi2, semg2, sems2),
            (colsv3, rowsv3, wv3, G3, M3, semi3, semg3, sems3))

    def zero_bufA():
        zz = jnp.zeros((L,), jnp.float32)
        def zb(i, c):
            for j in range(DH // L):
                bufA[i, pl.ds(j * L, L)] = zz
            return c
        lax.fori_loop(0, SBUF, zb, 0)

    def zero_xnew():
        for p in range(RPT // SBUF):
            pltpu.sync_copy(bufA, xnew.at[pl.ds(sstripe + p * SBUF, SBUF)])

    def start_idx(ci, p):
        colsv, rowsv, wv, _, _, semi, _, _ = bufs[p]
        base = ebase + ci * B
        pltpu.async_copy(cols.at[pl.ds(base, B)], colsv, semi)
        pltpu.async_copy(rows.at[pl.ds(base, B)], rowsv, semi)
        pltpu.async_copy(w.at[pl.ds(base, B)], wv, semi)

    def wait_idx(p):
        colsv, rowsv, wv, _, _, semi, _, _ = bufs[p]
        pltpu.make_async_copy(cols.at[pl.ds(0, B)], colsv, semi).wait()
        pltpu.make_async_copy(rows.at[pl.ds(0, B)], rowsv, semi).wait()
        pltpu.make_async_copy(w.at[pl.ds(0, B)], wv, semi).wait()

    def start_gather(src_hbm, p):
        colsv, _, _, G, _, _, semg, _ = bufs[p]
        wait_idx(p)
        for j in range(B // L):
            s = pl.ds(j * L, L)
            colsv[s] = colsv[s] + half_base
        pltpu.async_copy(src_hbm.at[colsv], G, semg)

    def wait_gather(src_hbm, p):
        colsv, _, _, G, _, _, semg, _ = bufs[p]
        pltpu.make_async_copy(src_hbm.at[colsv], G, semg).wait()

    def process(src_hbm, p):
        _, rowsv, wv, G, M, _, _, sems = bufs[p]
        wait_gather(src_hbm, p)
        UNR = 4
        def scale(i, c2):
            e0 = i * UNR
            base = jnp.zeros((L,), jnp.int32) + e0
            for t in range(UNR):
                wvec = plsc.load_gather(wv, [base + t])
                for j in range(DH // L):
                    s = pl.ds(j * L, L)
                    M[e0 + t, s] = G[e0 + t, s] * wvec
            return c2
        lax.fori_loop(0, B // UNR, scale, 0)
        pltpu.async_copy(M, xnew.at[rowsv], sems, add=True)

    def wait_scatter(p):
        _, rowsv, _, _, M, _, _, sems = bufs[p]
        pltpu.make_async_copy(M, xnew.at[rowsv], sems).wait()

    def edge_pass(src_hbm):
        start_idx(0, 0)
        start_idx(1, 1)
        start_gather(src_hbm, 0)
        start_gather(src_hbm, 1)
        process(src_hbm, 0)
        start_idx(2, 2)
        start_gather(src_hbm, 2)
        process(src_hbm, 1)
        start_idx(3, 3)

        def step(k, c):
            cb = 4 * k + 2
            for d in range(NBUF):
                p0 = (2 + d) % NBUF
                p1 = (3 + d) % NBUF
                p2 = (d) % NBUF
                start_gather(src_hbm, p1)
                process(src_hbm, p0)
                wait_scatter(p2)
                start_idx(cb + d + 2, p2)
            return c
        lax.fori_loop(0, (NCH - 4) // NBUF, step, 0)
        start_gather(src_hbm, (NCH - 1) % NBUF)
        process(src_hbm, (NCH - 2) % NBUF)
        wait_scatter(NCH % NBUF)
        process(src_hbm, (NCH - 1) % NBUF)
        wait_scatter((NCH + 1) % NBUF)
        wait_scatter((NCH - 2) % NBUF)
        wait_scatter((NCH - 1) % NBUF)

    zero_bufA()
    zero_xnew()
    plsc.subcore_barrier()
    edge_pass(xh)
    plsc.subcore_barrier()

    for p in range(RPT // SBUF):
        pltpu.sync_copy(xnew.at[pl.ds(sstripe + p * SBUF, SBUF)], bufB)
        pltpu.sync_copy(bufB, x1.at[pl.ds(stripe + p * SBUF, SBUF)])
    zero_xnew()
    plsc.subcore_barrier()

    edge_pass(x1)
    plsc.subcore_barrier()

    for p in range(RPT // SBUF):
        pltpu.sync_copy(xh.at[pl.ds(stripe + p * SBUF, SBUF)], bufA)
        pltpu.sync_copy(x1.at[pl.ds(stripe + p * SBUF, SBUF)], bufB)
        def addb(i, c):
            for j in range(DH // L):
                s = pl.ds(j * L, L)
                bufA[i, s] = bufA[i, s] + bufB[i, s]
            return c
        lax.fori_loop(0, SBUF, addb, 0)
        pltpu.sync_copy(xnew.at[pl.ds(sstripe + p * SBUF, SBUF)], bufB)
        def fin(i, c):
            for j in range(DH // L):
                s = pl.ds(j * L, L)
                bufA[i, s] = (bufA[i, s] + bufB[i, s]) * (1.0 / 3.0)
            return c
        lax.fori_loop(0, SBUF, fin, 0)
        pltpu.sync_copy(bufA, out.at[pl.ds(stripe + p * SBUF, SBUF)])


@jax.jit
def kernel(embedding, edge_index, edge_weight):
    xpad = jnp.pad(embedding, ((0, NPAD - N_NODE), (0, DPAD - EMB)))
    xh = jnp.concatenate([xpad[:, :DH], xpad[:, DH:]], axis=0)
    rows = edge_index[0]
    cols = edge_index[1]

    f32 = jnp.float32
    run = pl.kernel(
        _body,
        out_type=(
            jax.ShapeDtypeStruct((NC * NPAD, DH), f32),
            jax.ShapeDtypeStruct((NC * NPAD, DH), f32),
        ),
        mesh=plsc.VectorSubcoreMesh(
            core_axis_name="c", subcore_axis_name="s",
            num_cores=NC, num_subcores=NS),
        compiler_params=pltpu.CompilerParams(
            use_tc_tiling_on_sc=False, needs_layout_passes=False),
        scratch_types=(
            [pltpu.VMEM_SHARED((NPAD, DH), f32)]
            + [t for _ in range(NBUF)
               for t in (pltpu.VMEM((B,), jnp.int32),
                         pltpu.VMEM((B,), jnp.int32),
                         pltpu.VMEM((B,), f32),
                         pltpu.VMEM((B, DH), f32),
                         pltpu.VMEM((B, DH), f32))]
            + [pltpu.VMEM((SBUF, DH), f32),
               pltpu.VMEM((SBUF, DH), f32)]
            + [pltpu.SemaphoreType.DMA] * (3 * NBUF)
        ),
    )
    o, _x1 = run(xh, rows, cols, edge_weight)
    return jnp.concatenate([o[:N_NODE], o[NPAD:NPAD + N_NODE]], axis=1)[:, :EMB]


# speedup vs baseline: 1.8683x; 1.0019x over previous
import functools

import jax
import jax.numpy as jnp
from jax import lax
from jax.experimental import pallas as pl
from jax.experimental.pallas import tpu as pltpu, tpu_sc as plsc

N_NODE = 10000
NPAD = 10240
EMB = 112
DPAD = 128
DH = DPAD // 2
N_EDGE = 640000
NC = 2
NS = 16
L = 16
EPS = N_EDGE // NS
B = 80
NCH = EPS // B
RPT = NPAD // NS
SBUF = RPT // 2


NBUF = 4


def _body(xh, rows, cols, w, out, x1, xnew,
          colsv0, rowsv0, wv0, G0, M0, colsv1, rowsv1, wv1, G1, M1,
          colsv2, rowsv2, wv2, G2, M2, colsv3, rowsv3, wv3, G3, M3,
          bufA, bufB,
          semi0, semi1, semi2, semi3,
          semg0, semg1, semg2, semg3,
          sems0, sems1, sems2, sems3):
    cid = lax.axis_index("c")
    sid = lax.axis_index("s")
    half_base = cid * NPAD
    stripe = half_base + sid * RPT
    sstripe = sid * RPT
    ebase = sid * EPS

    bufs = ((colsv0, rowsv0, wv0, G0, M0, semi0, semg0, sems0),
            (colsv1, rowsv1, wv1, G1, M1, semi1, semg1, sems1),
            (colsv2, rowsv2, wv2, G2, M2, semi2, semg2, sems2),
            (colsv3, rowsv3, wv3, G3, M3, semi3, semg3, sems3))

    def zero_bufA():
        zz = jnp.zeros((L,), jnp.float32)
        def zb(i, c):
            for j in range(DH // L):
                bufA[i, pl.ds(j * L, L)] = zz
            return c
        lax.fori_loop(0, SBUF, zb, 0)

    def zero_xnew():
        for p in range(RPT // SBUF):
            pltpu.sync_copy(bufA, xnew.at[pl.ds(sstripe + p * SBUF, SBUF)])

    def start_idx(ci, p):
        colsv, rowsv, wv, _, _, semi, _, _ = bufs[p]
        base = ebase + ci * B
        pltpu.async_copy(cols.at[pl.ds(base, B)], colsv, semi)
        pltpu.async_copy(rows.at[pl.ds(base, B)], rowsv, semi)
        pltpu.async_copy(w.at[pl.ds(base, B)], wv, semi)

    def wait_idx(p):
        colsv, rowsv, wv, _, _, semi, _, _ = bufs[p]
        pltpu.make_async_copy(cols.at[pl.ds(0, B)], colsv, semi).wait()
        pltpu.make_async_copy(rows.at[pl.ds(0, B)], rowsv, semi).wait()
        pltpu.make_async_copy(w.at[pl.ds(0, B)], wv, semi).wait()

    def start_gather(src_hbm, p):
        colsv, _, _, G, _, _, semg, _ = bufs[p]
        wait_idx(p)
        for j in range(B // L):
            s = pl.ds(j * L, L)
            colsv[s] = colsv[s] + half_base
        pltpu.async_copy(src_hbm.at[colsv], G, semg)

    def wait_gather(src_hbm, p):
        colsv, _, _, G, _, _, semg, _ = bufs[p]
        pltpu.make_async_copy(src_hbm.at[colsv], G, semg).wait()

    def process(src_hbm, p):
        _, rowsv, wv, G, M, _, _, sems = bufs[p]
        wait_gather(src_hbm, p)
        UNR = 8
        NJ = DH // L
        def scale(i, c2):
            e0 = i * UNR
            base = jnp.zeros((L,), jnp.int32) + e0
            wts = [plsc.load_gather(wv, [base + t]) for t in range(UNR)]
            gs = [[G[e0 + t, pl.ds(j * L, L)] for j in range(NJ)]
                  for t in range(UNR)]
            ms = [[gs[t][j] * wts[t] for j in range(NJ)] for t in range(UNR)]
            for t in range(UNR):
                for j in range(NJ):
                    M[e0 + t, pl.ds(j * L, L)] = ms[t][j]
            return c2
        lax.fori_loop(0, B // UNR, scale, 0)
        pltpu.async_copy(M, xnew.at[rowsv], sems, add=True)

    def wait_scatter(p):
        _, rowsv, _, _, M, _, _, sems = bufs[p]
        pltpu.make_async_copy(M, xnew.at[rowsv], sems).wait()

    def edge_pass(src_hbm):
        start_idx(0, 0)
        start_idx(1, 1)
        start_gather(src_hbm, 0)
        start_gather(src_hbm, 1)
        process(src_hbm, 0)
        start_idx(2, 2)
        start_gather(src_hbm, 2)
        process(src_hbm, 1)
        start_idx(3, 3)

        def step(k, c):
            cb = 4 * k + 2
            for d in range(NBUF):
                p0 = (2 + d) % NBUF
                p1 = (3 + d) % NBUF
                p2 = (d) % NBUF
                start_gather(src_hbm, p1)
                process(src_hbm, p0)
                wait_scatter(p2)
                start_idx(cb + d + 2, p2)
            return c
        lax.fori_loop(0, (NCH - 4) // NBUF, step, 0)
        start_gather(src_hbm, (NCH - 1) % NBUF)
        process(src_hbm, (NCH - 2) % NBUF)
        wait_scatter(NCH % NBUF)
        process(src_hbm, (NCH - 1) % NBUF)
        wait_scatter((NCH + 1) % NBUF)
        wait_scatter((NCH - 2) % NBUF)
        wait_scatter((NCH - 1) % NBUF)

    zero_bufA()
    zero_xnew()
    plsc.subcore_barrier()
    edge_pass(xh)
    plsc.subcore_barrier()

    for p in range(RPT // SBUF):
        pltpu.sync_copy(xnew.at[pl.ds(sstripe + p * SBUF, SBUF)], bufB)
        pltpu.sync_copy(bufB, x1.at[pl.ds(stripe + p * SBUF, SBUF)])
    zero_xnew()
    plsc.subcore_barrier()

    edge_pass(x1)
    plsc.subcore_barrier()

    for p in range(RPT // SBUF):
        pltpu.sync_copy(xh.at[pl.ds(stripe + p * SBUF, SBUF)], bufA)
        pltpu.sync_copy(x1.at[pl.ds(stripe + p * SBUF, SBUF)], bufB)
        def addb(i, c):
            for j in range(DH // L):
                s = pl.ds(j * L, L)
                bufA[i, s] = bufA[i, s] + bufB[i, s]
            return c
        lax.fori_loop(0, SBUF, addb, 0)
        pltpu.sync_copy(xnew.at[pl.ds(sstripe + p * SBUF, SBUF)], bufB)
        def fin(i, c):
            for j in range(DH // L):
                s = pl.ds(j * L, L)
                bufA[i, s] = (bufA[i, s] + bufB[i, s]) * (1.0 / 3.0)
            return c
        lax.fori_loop(0, SBUF, fin, 0)
        pltpu.sync_copy(bufA, out.at[pl.ds(stripe + p * SBUF, SBUF)])


@jax.jit
def kernel(embedding, edge_index, edge_weight):
    xpad = jnp.pad(embedding, ((0, NPAD - N_NODE), (0, DPAD - EMB)))
    xh = jnp.concatenate([xpad[:, :DH], xpad[:, DH:]], axis=0)
    rows = edge_index[0]
    cols = edge_index[1]

    f32 = jnp.float32
    run = pl.kernel(
        _body,
        out_type=(
            jax.ShapeDtypeStruct((NC * NPAD, DH), f32),
            jax.ShapeDtypeStruct((NC * NPAD, DH), f32),
        ),
        mesh=plsc.VectorSubcoreMesh(
            core_axis_name="c", subcore_axis_name="s",
            num_cores=NC, num_subcores=NS),
        compiler_params=pltpu.CompilerParams(
            use_tc_tiling_on_sc=False, needs_layout_passes=False),
        scratch_types=(
            [pltpu.VMEM_SHARED((NPAD, DH), f32)]
            + [t for _ in range(NBUF)
               for t in (pltpu.VMEM((B,), jnp.int32),
                         pltpu.VMEM((B,), jnp.int32),
                         pltpu.VMEM((B,), f32),
                         pltpu.VMEM((B, DH), f32),
                         pltpu.VMEM((B, DH), f32))]
            + [pltpu.VMEM((SBUF, DH), f32),
               pltpu.VMEM((SBUF, DH), f32)]
            + [pltpu.SemaphoreType.DMA] * (3 * NBUF)
        ),
    )
    o, _x1 = run(xh, rows, cols, edge_weight)
    return jnp.concatenate([o[:N_NODE], o[NPAD:NPAD + N_NODE]], axis=1)[:, :EMB]
